# Initial kernel scaffold; baseline (speedup 1.0000x reference)
import jax
import jax.numpy as jnp
from jax.experimental import pallas as pl


def kernel(x, edge_index, batch, W1_0, b1_0, W2_0, b2_0, g_0, be_0, W1_1, b1_1, W2_1, b2_1, g_1, be_1, W1_2, b1_2, W2_2, b2_2, g_2, be_2, fc1_w, fc1_b, fc2_w, fc2_b):
    raise NotImplementedError("write your pallas kernel here")


# speedup vs baseline: 9.8104x; 9.8104x over previous
import functools

import jax
import jax.numpy as jnp
from jax import lax
from jax.experimental import pallas as pl
from jax.experimental.pallas import tpu as pltpu
from jax.experimental.pallas import tpu_sc as plsc

_NC = 2
_NS = 16


@functools.partial(jax.jit, static_argnames=("n", "h", "nch", "chunk"))
def _sc_edge_agg(p, src3, dst3, *, n, h, nch, chunk):
    npad = -(-n // (8 * _NS)) * (8 * _NS)
    rows_per_tile = npad // _NS
    mesh = plsc.VectorSubcoreMesh(core_axis_name="c", subcore_axis_name="s")

    @functools.partial(
        pl.kernel,
        out_type=jax.ShapeDtypeStruct((_NC, npad, h), jnp.float32),
        mesh=mesh,
        scratch_types=[
            pltpu.VMEM((nch, chunk), jnp.int32),
            pltpu.VMEM((nch, chunk), jnp.int32),
            pltpu.VMEM((chunk, h), jnp.float32),
            pltpu.VMEM((rows_per_tile, h), jnp.float32),
            pltpu.VMEM_SHARED((npad, h), jnp.float32),
            pltpu.SemaphoreType.DMA,
        ],
        compiler_params=pltpu.CompilerParams(use_tc_tiling_on_sc=False),
    )
    def body(p_hbm, src_hbm, dst_hbm, out_hbm, idxs_v, idxd_v, rows_v,
             slab_v, acc_sh, sem):
        c = lax.axis_index("c")
        s = lax.axis_index("s")
        w = c * _NS + s

        zeros16 = jnp.zeros((16,), jnp.float32)

        def zero_row(r, carry):
            for lo in range(0, h, 16):
                slab_v[r, pl.ds(lo, 16)] = zeros16
            return carry

        lax.fori_loop(0, rows_per_tile, zero_row, 0)
        pltpu.sync_copy(slab_v, acc_sh.at[pl.ds(s * rows_per_tile,
                                                rows_per_tile)])

        pltpu.sync_copy(src_hbm.at[w], idxs_v)
        pltpu.sync_copy(dst_hbm.at[w], idxd_v)
        plsc.subcore_barrier()

        def edge_chunk(j, carry):
            pltpu.async_copy(p_hbm.at[idxs_v.at[j]], rows_v, sem).wait()
            pltpu.sync_copy(rows_v, acc_sh.at[idxd_v.at[j]], add=True)
            return carry

        lax.fori_loop(0, nch, edge_chunk, 0)
        plsc.subcore_barrier()

        pltpu.sync_copy(acc_sh.at[pl.ds(s * rows_per_tile, rows_per_tile)],
                        slab_v)
        pltpu.sync_copy(
            slab_v,
            out_hbm.at[c].at[pl.ds(s * rows_per_tile, rows_per_tile)])

    return body(p, src3, dst3)


def _proj_body(x_ref, w_ref, o_ref):
    o_ref[...] = jnp.dot(x_ref[...], w_ref[...],
                         preferred_element_type=jnp.float32)


def _post(p_ref, a_ref, b1_ref, w2_ref, b2_ref, g_ref, be_ref):
    z = p_ref[...] + a_ref[0] + a_ref[1] + b1_ref[...]
    z = jnp.maximum(z, 0.0)
    z = jnp.dot(z, w2_ref[...], preferred_element_type=jnp.float32)
    z = jnp.maximum(z + b2_ref[...], 0.0)
    return z * g_ref[...] + be_ref[...]


def _mid_body(p_ref, a_ref, b1_ref, w2_ref, b2_ref, g_ref, be_ref, w1n_ref,
              o_ref):
    hcur = _post(p_ref, a_ref, b1_ref, w2_ref, b2_ref, g_ref, be_ref)
    o_ref[...] = jnp.dot(hcur, w1n_ref[...],
                         preferred_element_type=jnp.float32)


def _final_body(p_ref, a_ref, b1_ref, w2_ref, b2_ref, g_ref, be_ref,
                batch_ref, fc1w_ref, fc1b_ref, fc2w_ref, fc2b_ref,
                o_ref, acc_ref, *, n_graphs):
    i = pl.program_id(0)
    hcur = _post(p_ref, a_ref, b1_ref, w2_ref, b2_ref, g_ref, be_ref)
    blk = hcur.shape[0]
    onehot = (batch_ref[0, 0, :][:, None]
              == lax.broadcasted_iota(jnp.int32, (blk, n_graphs), 1)
              ).astype(jnp.float32)
    part = lax.dot_general(onehot, hcur, (((0,), (0,)), ((), ())),
                           preferred_element_type=jnp.float32)

    @pl.when(i == 0)
    def _():
        acc_ref[...] = part

    @pl.when(i > 0)
    def _():
        acc_ref[...] += part

    @pl.when(i == pl.num_programs(0) - 1)
    def _():
        u = jnp.dot(acc_ref[...], fc1w_ref[...],
                    preferred_element_type=jnp.float32) + fc1b_ref[...]
        u = jnp.maximum(u, 0.0)
        u = jnp.dot(u, fc2w_ref[...],
                    preferred_element_type=jnp.float32) + fc2b_ref[...]
        m = jnp.mean(u, axis=0, keepdims=True)
        mx = jnp.max(m)
        e = jnp.exp(m - mx)
        o_ref[...] = m - mx - jnp.log(jnp.sum(e))


def _full(shape):
    return pl.BlockSpec(shape, lambda i: (0,) * len(shape))


def kernel(x, edge_index, batch, W1_0, b1_0, W2_0, b2_0, g_0, be_0,
           W1_1, b1_1, W2_1, b2_1, g_1, be_1, W1_2, b1_2, W2_2, b2_2,
           g_2, be_2, fc1_w, fc1_b, fc2_w, fc2_b):
    n, d = x.shape
    h = W1_0.shape[1]
    n_graphs = 128
    c = fc2_w.shape[1]
    e = edge_index.shape[1]

    blk = 2000
    nb = n // blk

    nw = _NC * _NS
    ew = e // nw
    chunk = max(ck for ck in range(8, 129, 8) if ew % ck == 0)
    nch = ew // chunk
    src3 = edge_index[0].reshape(nw, nch, chunk)
    dst3 = edge_index[1].reshape(nw, nch, chunk)

    b1s = [b1_0.reshape(1, h), b1_1.reshape(1, h), b1_2.reshape(1, h)]
    b2s = [b2_0.reshape(1, h), b2_1.reshape(1, h), b2_2.reshape(1, h)]
    gs = [g_0.reshape(1, h), g_1.reshape(1, h), g_2.reshape(1, h)]
    bes = [be_0.reshape(1, h), be_1.reshape(1, h), be_2.reshape(1, h)]
    w2s = [W2_0, W2_1, W2_2]
    batch3 = batch.reshape(nb, 1, blk)

    p = pl.pallas_call(
        _proj_body,
        grid=(nb,),
        in_specs=[pl.BlockSpec((blk, d), lambda i: (i, 0)),
                  _full((d, h))],
        out_specs=pl.BlockSpec((blk, h), lambda i: (i, 0)),
        out_shape=jax.ShapeDtypeStruct((n, h), jnp.float32),
    )(x, W1_0)

    vec_spec = [pl.BlockSpec((blk, h), lambda i: (i, 0)),
                pl.BlockSpec((_NC, blk, h), lambda i: (0, i, 0))]
    small = [_full((1, h)), _full((h, h)), _full((1, h)), _full((1, h)),
             _full((1, h))]

    for layer in range(2):
        agg = _sc_edge_agg(p, src3, dst3, n=n, h=h, nch=nch, chunk=chunk)
        p = pl.pallas_call(
            _mid_body,
            grid=(nb,),
            in_specs=vec_spec + small + [_full((h, h))],
            out_specs=pl.BlockSpec((blk, h), lambda i: (i, 0)),
            out_shape=jax.ShapeDtypeStruct((n, h), jnp.float32),
        )(p, agg, b1s[layer], w2s[layer], b2s[layer], gs[layer], bes[layer],
          [W1_1, W1_2][layer])

    agg = _sc_edge_agg(p, src3, dst3, n=n, h=h, nch=nch, chunk=chunk)
    out = pl.pallas_call(
        functools.partial(_final_body, n_graphs=n_graphs),
        grid=(nb,),
        in_specs=vec_spec + small
        + [pl.BlockSpec((1, 1, blk), lambda i: (i, 0, 0)),
           _full((h, h)), _full((1, h)), _full((h, c)), _full((1, c))],
        out_specs=_full((1, c)),
        out_shape=jax.ShapeDtypeStruct((1, c), jnp.float32),
        scratch_shapes=[pltpu.VMEM((n_graphs, h), jnp.float32)],
    )(p, agg, b1s[2], w2s[2], b2s[2], gs[2], bes[2], batch3,
      fc1_w, fc1_b.reshape(1, h), fc2_w, fc2_b.reshape(1, c))

    return out.reshape(c)


# speedup vs baseline: 21.7121x; 2.2132x over previous
import functools

import jax
import jax.numpy as jnp
from jax import lax
from jax.experimental import pallas as pl
from jax.experimental.pallas import tpu as pltpu
from jax.experimental.pallas import tpu_sc as plsc

_NC = 2
_NS = 16


@functools.partial(jax.jit, static_argnames=("n", "h", "nch", "chunk"))
def _sc_edge_agg(p, src3, dst3, *, n, h, nch, chunk):
    npad = -(-n // (8 * _NS)) * (8 * _NS)
    rows_per_tile = npad // _NS
    mesh = plsc.VectorSubcoreMesh(core_axis_name="c", subcore_axis_name="s")
    nb = 5
    assert nch % nb == 0
    ngroups = nch // nb

    @functools.partial(
        pl.kernel,
        out_type=jax.ShapeDtypeStruct((_NC, npad, h), jnp.float32),
        mesh=mesh,
        scratch_types=[
            pltpu.VMEM((nch, chunk), jnp.int32),
            pltpu.VMEM((nch, chunk), jnp.int32),
            pltpu.VMEM((nb, chunk, h), jnp.float32),
            pltpu.VMEM((rows_per_tile, h), jnp.float32),
            pltpu.VMEM_SHARED((npad, h), jnp.float32),
            pltpu.SemaphoreType.DMA((nb,)),
        ],
        compiler_params=pltpu.CompilerParams(use_tc_tiling_on_sc=False),
    )
    def body(p_hbm, src_hbm, dst_hbm, out_hbm, idxs_v, idxd_v, rows_v,
             slab_v, acc_sh, sems):
        c = lax.axis_index("c")
        s = lax.axis_index("s")
        w = c * _NS + s

        pltpu.sync_copy(src_hbm.at[w], idxs_v)
        pltpu.sync_copy(dst_hbm.at[w], idxd_v)

        zeros16 = jnp.zeros((16,), jnp.float32)

        def zero_rows(r, carry):
            for rr in range(4):
                for lo in range(0, h, 16):
                    slab_v[r * 4 + rr, pl.ds(lo, 16)] = zeros16
            return carry

        lax.fori_loop(0, rows_per_tile // 4, zero_rows, 0)
        pltpu.sync_copy(slab_v, acc_sh.at[pl.ds(s * rows_per_tile,
                                                rows_per_tile)])
        plsc.subcore_barrier()

        def wait_gather(b):
            pltpu.make_async_copy(p_hbm.at[idxs_v.at[0]], rows_v.at[b],
                                  sems.at[b]).wait()

        for b in range(nb):
            pltpu.async_copy(p_hbm.at[idxs_v.at[b]], rows_v.at[b],
                             sems.at[b])

        def group(g, carry):
            for b in range(nb):
                j = g * nb + b
                wait_gather(b)
                pltpu.sync_copy(rows_v.at[b], acc_sh.at[idxd_v.at[j]],
                                add=True)
                pltpu.async_copy(p_hbm.at[idxs_v.at[j + nb]], rows_v.at[b],
                                 sems.at[b])
            return carry

        lax.fori_loop(0, ngroups - 1, group, 0)
        for b in range(nb):
            j = (ngroups - 1) * nb + b
            wait_gather(b)
            pltpu.sync_copy(rows_v.at[b], acc_sh.at[idxd_v.at[j]], add=True)
        plsc.subcore_barrier()

        pltpu.sync_copy(acc_sh.at[pl.ds(s * rows_per_tile, rows_per_tile)],
                        slab_v)
        pltpu.sync_copy(
            slab_v,
            out_hbm.at[c].at[pl.ds(s * rows_per_tile, rows_per_tile)])

    return body(p, src3, dst3)


def _proj_body(x_ref, w_ref, o_ref):
    o_ref[...] = jnp.dot(x_ref[...], w_ref[...],
                         preferred_element_type=jnp.float32)


def _post(p_ref, a_ref, b1_ref, w2_ref, b2_ref, g_ref, be_ref):
    z = p_ref[...] + a_ref[0] + a_ref[1] + b1_ref[...]
    z = jnp.maximum(z, 0.0)
    z = jnp.dot(z, w2_ref[...], preferred_element_type=jnp.float32)
    z = jnp.maximum(z + b2_ref[...], 0.0)
    return z * g_ref[...] + be_ref[...]


def _mid_body(p_ref, a_ref, b1_ref, w2_ref, b2_ref, g_ref, be_ref, w1n_ref,
              o_ref):
    hcur = _post(p_ref, a_ref, b1_ref, w2_ref, b2_ref, g_ref, be_ref)
    o_ref[...] = jnp.dot(hcur, w1n_ref[...],
                         preferred_element_type=jnp.float32)


def _final_body(p_ref, a_ref, b1_ref, w2_ref, b2_ref, g_ref, be_ref,
                batch_ref, fc1w_ref, fc1b_ref, fc2w_ref, fc2b_ref,
                o_ref, acc_ref, *, n_graphs):
    i = pl.program_id(0)
    hcur = _post(p_ref, a_ref, b1_ref, w2_ref, b2_ref, g_ref, be_ref)
    blk = hcur.shape[0]
    onehot = (batch_ref[0, 0, :][:, None]
              == lax.broadcasted_iota(jnp.int32, (blk, n_graphs), 1)
              ).astype(jnp.float32)
    part = lax.dot_general(onehot, hcur, (((0,), (0,)), ((), ())),
                           preferred_element_type=jnp.float32)

    @pl.when(i == 0)
    def _():
        acc_ref[...] = part

    @pl.when(i > 0)
    def _():
        acc_ref[...] += part

    @pl.when(i == pl.num_programs(0) - 1)
    def _():
        u = jnp.dot(acc_ref[...], fc1w_ref[...],
                    preferred_element_type=jnp.float32) + fc1b_ref[...]
        u = jnp.maximum(u, 0.0)
        u = jnp.dot(u, fc2w_ref[...],
                    preferred_element_type=jnp.float32) + fc2b_ref[...]
        m = jnp.mean(u, axis=0, keepdims=True)
        mx = jnp.max(m)
        e = jnp.exp(m - mx)
        o_ref[...] = m - mx - jnp.log(jnp.sum(e))


def _full(shape):
    return pl.BlockSpec(shape, lambda i: (0,) * len(shape))


def kernel(x, edge_index, batch, W1_0, b1_0, W2_0, b2_0, g_0, be_0,
           W1_1, b1_1, W2_1, b2_1, g_1, be_1, W1_2, b1_2, W2_2, b2_2,
           g_2, be_2, fc1_w, fc1_b, fc2_w, fc2_b):
    n, d = x.shape
    h = W1_0.shape[1]
    n_graphs = 128
    c = fc2_w.shape[1]
    e = edge_index.shape[1]

    blk = 2000
    nb = n // blk

    nw = _NC * _NS
    ew = e // nw
    chunk = max(ck for ck in range(8, 129, 8) if ew % ck == 0)
    nch = ew // chunk
    src3 = edge_index[0].reshape(nw, nch, chunk)
    dst3 = edge_index[1].reshape(nw, nch, chunk)

    b1s = [b1_0.reshape(1, h), b1_1.reshape(1, h), b1_2.reshape(1, h)]
    b2s = [b2_0.reshape(1, h), b2_1.reshape(1, h), b2_2.reshape(1, h)]
    gs = [g_0.reshape(1, h), g_1.reshape(1, h), g_2.reshape(1, h)]
    bes = [be_0.reshape(1, h), be_1.reshape(1, h), be_2.reshape(1, h)]
    w2s = [W2_0, W2_1, W2_2]
    batch3 = batch.reshape(nb, 1, blk)

    p = pl.pallas_call(
        _proj_body,
        grid=(nb,),
        in_specs=[pl.BlockSpec((blk, d), lambda i: (i, 0)),
                  _full((d, h))],
        out_specs=pl.BlockSpec((blk, h), lambda i: (i, 0)),
        out_shape=jax.ShapeDtypeStruct((n, h), jnp.float32),
    )(x, W1_0)

    vec_spec = [pl.BlockSpec((blk, h), lambda i: (i, 0)),
                pl.BlockSpec((_NC, blk, h), lambda i: (0, i, 0))]
    small = [_full((1, h)), _full((h, h)), _full((1, h)), _full((1, h)),
             _full((1, h))]

    for layer in range(2):
        agg = _sc_edge_agg(p, src3, dst3, n=n, h=h, nch=nch, chunk=chunk)
        p = pl.pallas_call(
            _mid_body,
            grid=(nb,),
            in_specs=vec_spec + small + [_full((h, h))],
            out_specs=pl.BlockSpec((blk, h), lambda i: (i, 0)),
            out_shape=jax.ShapeDtypeStruct((n, h), jnp.float32),
        )(p, agg, b1s[layer], w2s[layer], b2s[layer], gs[layer], bes[layer],
          [W1_1, W1_2][layer])

    agg = _sc_edge_agg(p, src3, dst3, n=n, h=h, nch=nch, chunk=chunk)
    out = pl.pallas_call(
        functools.partial(_final_body, n_graphs=n_graphs),
        grid=(nb,),
        in_specs=vec_spec + small
        + [pl.BlockSpec((1, 1, blk), lambda i: (i, 0, 0)),
           _full((h, h)), _full((1, h)), _full((h, c)), _full((1, c))],
        out_specs=_full((1, c)),
        out_shape=jax.ShapeDtypeStruct((1, c), jnp.float32),
        scratch_shapes=[pltpu.VMEM((n_graphs, h), jnp.float32)],
    )(p, agg, b1s[2], w2s[2], b2s[2], gs[2], bes[2], batch3,
      fc1_w, fc1_b.reshape(1, h), fc2_w, fc2_b.reshape(1, c))

    return out.reshape(c)
